# initial kernel scaffold (unmeasured)
import functools

import jax
import jax.numpy as jnp
from jax import lax
from jax.experimental import pallas as pl
from jax.experimental.pallas import tpu as pltpu

N_DEV = 4
SQ = 512
D = 1024
DH = 128
HQ_LOCAL = 8
GROUP = 4
KV_LOCAL = HQ_LOCAL // GROUP
SCALE = 0.08838834764831843


def _body(x_ref, wq_ref, wo_ref, wk_ref, wv_ref, out_ref,
          attn_ref, comm_ref, send_sems, recv_sems):
    my_pos = lax.axis_index("i")
    left = (my_pos - 1) % N_DEV
    right = (my_pos + 1) % N_DEV

    barrier_sem = pltpu.get_barrier_semaphore()
    for nbr in [left, right]:
        pl.semaphore_signal(
            barrier_sem, inc=1,
            device_id=(nbr,), device_id_type=pl.DeviceIdType.MESH,
        )
    pl.semaphore_wait(barrier_sem, 2)

    xb = x_ref[0].astype(jnp.bfloat16)
    wq = wq_ref[...].astype(jnp.bfloat16)
    kv_col = my_pos * (KV_LOCAL * DH)
    wk = wk_ref[:, pl.ds(kv_col, KV_LOCAL * DH)].astype(jnp.bfloat16)
    wv = wv_ref[:, pl.ds(kv_col, KV_LOCAL * DH)].astype(jnp.bfloat16)

    q = jnp.dot(xb, wq, preferred_element_type=jnp.float32).astype(jnp.bfloat16)
    k = jnp.dot(xb, wk, preferred_element_type=jnp.float32).astype(jnp.bfloat16)
    v = jnp.dot(xb, wv, preferred_element_type=jnp.float32).astype(jnp.bfloat16)

    for h in range(HQ_LOCAL):
        qh = q[:, h * DH:(h + 1) * DH]
        kvh = h // GROUP
        kh = k[:, kvh * DH:(kvh + 1) * DH]
        vh = v[:, kvh * DH:(kvh + 1) * DH]
        s = lax.dot_general(
            qh, kh, (((1,), (1,)), ((), ())),
            preferred_element_type=jnp.float32,
        ) * SCALE
        m = jnp.max(s, axis=1, keepdims=True)
        p = jnp.exp(s - m)
        l = jnp.sum(p, axis=1, keepdims=True)
        o = jnp.dot(p.astype(jnp.bfloat16), vh,
                    preferred_element_type=jnp.float32)
        attn_ref[:, h * DH:(h + 1) * DH] = (o / l).astype(jnp.bfloat16)

    wo = wo_ref[...].astype(jnp.bfloat16)
    y = jnp.dot(attn_ref[...], wo, preferred_element_type=jnp.float32)

    out_ref[...] = y
    comm_ref[0] = y

    for hop in range(N_DEV - 1):
        send_slot = hop % 2
        recv_slot = (hop + 1) % 2
        rdma = pltpu.make_async_remote_copy(
            src_ref=comm_ref.at[send_slot],
            dst_ref=comm_ref.at[recv_slot],
            send_sem=send_sems.at[send_slot],
            recv_sem=recv_sems.at[recv_slot],
            device_id=(right,),
            device_id_type=pl.DeviceIdType.MESH,
        )
        rdma.start()
        rdma.wait()
        out_ref[...] += comm_ref[recv_slot]


def kernel(x, Wq, Wo, Wk, Wv):
    x2 = x.reshape(SQ, D)
    out = pl.pallas_call(
        _body,
        out_shape=jax.ShapeDtypeStruct((SQ, D), jnp.float32),
        in_specs=[pl.BlockSpec(memory_space=pltpu.VMEM)] * 5,
        out_specs=pl.BlockSpec(memory_space=pltpu.VMEM),
        scratch_shapes=[
            pltpu.VMEM((SQ, D), jnp.bfloat16),
            pltpu.VMEM((2, SQ, D), jnp.float32),
            pltpu.SemaphoreType.DMA((2,)),
            pltpu.SemaphoreType.DMA((2,)),
        ],
        compiler_params=pltpu.CompilerParams(collective_id=0),
    )(x2, Wq, Wo, Wk, Wv)
    return out.reshape(1, SQ, D)


# baseline (device time: 94238 ns/iter reference)
import functools

import jax
import jax.numpy as jnp
from jax import lax
from jax.experimental import pallas as pl
from jax.experimental.pallas import tpu as pltpu

N_DEV = 4
SQ = 512
D = 1024
DH = 128
HQ_LOCAL = 8
GROUP = 4
KV_LOCAL = HQ_LOCAL // GROUP
SCALE = 0.08838834764831843


def _body(x_ref, wq_ref, wo_ref, wk_ref, wv_ref, out_ref,
          attn_ref, comm_ref, send_sems, recv_sems):
    my_pos = lax.axis_index("i")
    left = (my_pos - 1) % N_DEV
    right = (my_pos + 1) % N_DEV

    barrier_sem = pltpu.get_barrier_semaphore()
    for nbr in [left, right]:
        pl.semaphore_signal(
            barrier_sem, inc=1,
            device_id=(nbr,), device_id_type=pl.DeviceIdType.MESH,
        )
    pl.semaphore_wait(barrier_sem, 2)

    xb = x_ref[...].astype(jnp.bfloat16)
    wq = wq_ref[...].astype(jnp.bfloat16)
    kv_col = my_pos * (KV_LOCAL * DH)
    wk = wk_ref[:, pl.ds(kv_col, KV_LOCAL * DH)].astype(jnp.bfloat16)
    wv = wv_ref[:, pl.ds(kv_col, KV_LOCAL * DH)].astype(jnp.bfloat16)

    q = jnp.dot(xb, wq, preferred_element_type=jnp.float32).astype(jnp.bfloat16)
    k = jnp.dot(xb, wk, preferred_element_type=jnp.float32).astype(jnp.bfloat16)
    v = jnp.dot(xb, wv, preferred_element_type=jnp.float32).astype(jnp.bfloat16)

    for h in range(HQ_LOCAL):
        qh = q[:, h * DH:(h + 1) * DH]
        kvh = h // GROUP
        kh = k[:, kvh * DH:(kvh + 1) * DH]
        vh = v[:, kvh * DH:(kvh + 1) * DH]
        s = lax.dot_general(
            qh, kh, (((1,), (1,)), ((), ())),
            preferred_element_type=jnp.float32,
        ) * SCALE
        m = jnp.max(s, axis=1, keepdims=True)
        p = jnp.exp(s - m)
        l = jnp.sum(p, axis=1, keepdims=True)
        o = jnp.dot(p.astype(jnp.bfloat16), vh,
                    preferred_element_type=jnp.float32)
        attn_ref[:, h * DH:(h + 1) * DH] = (o / l).astype(jnp.bfloat16)

    wo = wo_ref[...].astype(jnp.bfloat16)
    y = jnp.dot(attn_ref[...], wo, preferred_element_type=jnp.float32)

    out_ref[...] = y
    comm_ref[0] = y

    for hop in range(N_DEV - 1):
        send_slot = hop % 2
        recv_slot = (hop + 1) % 2
        rdma = pltpu.make_async_remote_copy(
            src_ref=comm_ref.at[send_slot],
            dst_ref=comm_ref.at[recv_slot],
            send_sem=send_sems.at[send_slot],
            recv_sem=recv_sems.at[recv_slot],
            device_id=(right,),
            device_id_type=pl.DeviceIdType.MESH,
        )
        rdma.start()
        rdma.wait()
        out_ref[...] += comm_ref[recv_slot]


def kernel(x, Wq, Wo, Wk, Wv):
    x2 = x.reshape(SQ, D)
    out = pl.pallas_call(
        _body,
        out_shape=jax.ShapeDtypeStruct((SQ, D), jnp.float32),
        in_specs=[pl.BlockSpec(memory_space=pltpu.VMEM)] * 5,
        out_specs=pl.BlockSpec(memory_space=pltpu.VMEM),
        scratch_shapes=[
            pltpu.VMEM((SQ, D), jnp.bfloat16),
            pltpu.VMEM((2, SQ, D), jnp.float32),
            pltpu.SemaphoreType.DMA((2,)),
            pltpu.SemaphoreType.DMA((2,)),
        ],
        compiler_params=pltpu.CompilerParams(collective_id=0),
    )(x2, Wq, Wo, Wk, Wv)
    return out.reshape(1, SQ, D)


# device time: 43667 ns/iter; 2.1581x vs baseline; 2.1581x over previous
import jax
import jax.numpy as jnp
from jax import lax
from jax.experimental import pallas as pl
from jax.experimental.pallas import tpu as pltpu

N_DEV = 4
SQ = 512
D = 1024
HALF = D // 2
DH = 128
HQ_LOCAL = 8
GROUP = 4
KV_LOCAL = HQ_LOCAL // GROUP
SCALE = 0.08838834764831843


def _body(x_ref, wq_ref, wo_ref, wk_ref, wv_ref, out_ref,
          attn_ref, comm_r, comm_l, sr_send, sr_recv, sl_send, sl_recv):
    my_pos = lax.axis_index("i")
    left = (my_pos - 1) % N_DEV
    right = (my_pos + 1) % N_DEV

    barrier_sem = pltpu.get_barrier_semaphore()
    for nbr in [left, right]:
        pl.semaphore_signal(
            barrier_sem, inc=1,
            device_id=(nbr,), device_id_type=pl.DeviceIdType.MESH,
        )
    pl.semaphore_wait(barrier_sem, 2)

    xb = x_ref[...].astype(jnp.bfloat16)
    wq = wq_ref[...].astype(jnp.bfloat16)
    kv_col = my_pos * (KV_LOCAL * DH)
    wk = wk_ref[:, pl.ds(kv_col, KV_LOCAL * DH)].astype(jnp.bfloat16)
    wv = wv_ref[:, pl.ds(kv_col, KV_LOCAL * DH)].astype(jnp.bfloat16)

    q = jnp.dot(xb, wq, preferred_element_type=jnp.float32).astype(jnp.bfloat16)
    k = jnp.dot(xb, wk, preferred_element_type=jnp.float32).astype(jnp.bfloat16)
    v = jnp.dot(xb, wv, preferred_element_type=jnp.float32).astype(jnp.bfloat16)

    for h in range(HQ_LOCAL):
        qh = q[:, h * DH:(h + 1) * DH]
        kvh = h // GROUP
        kh = k[:, kvh * DH:(kvh + 1) * DH]
        vh = v[:, kvh * DH:(kvh + 1) * DH]
        s = lax.dot_general(
            qh, kh, (((1,), (1,)), ((), ())),
            preferred_element_type=jnp.float32,
        ) * SCALE
        m = jnp.max(s, axis=1, keepdims=True)
        p = jnp.exp(s - m)
        l = jnp.sum(p, axis=1, keepdims=True)
        o = jnp.dot(p.astype(jnp.bfloat16), vh,
                    preferred_element_type=jnp.float32)
        attn_ref[:, h * DH:(h + 1) * DH] = (o / l).astype(jnp.bfloat16)

    attn = attn_ref[...]

    def make_rdma(dir_comm, dir_send, dir_recv, hop, dev):
        return pltpu.make_async_remote_copy(
            src_ref=dir_comm.at[hop],
            dst_ref=dir_comm.at[hop + 1],
            send_sem=dir_send.at[hop],
            recv_sem=dir_recv.at[hop],
            device_id=(dev,),
            device_id_type=pl.DeviceIdType.MESH,
        )

    y_l = jnp.dot(attn, wo_ref[:, :HALF].astype(jnp.bfloat16),
                  preferred_element_type=jnp.float32)
    comm_r[0] = y_l.astype(jnp.bfloat16)
    rd_r = make_rdma(comm_r, sr_send, sr_recv, 0, right)
    rd_r.start()

    y_r = jnp.dot(attn, wo_ref[:, HALF:].astype(jnp.bfloat16),
                  preferred_element_type=jnp.float32)
    comm_l[0] = y_r.astype(jnp.bfloat16)
    rd_l = make_rdma(comm_l, sl_send, sl_recv, 0, left)
    rd_l.start()

    out_ref[:, :HALF] = y_l
    out_ref[:, HALF:] = y_r

    for hop in range(N_DEV - 1):
        rd_r.wait()
        rd_l.wait()
        if hop < N_DEV - 2:
            rd_r = make_rdma(comm_r, sr_send, sr_recv, hop + 1, right)
            rd_r.start()
            rd_l = make_rdma(comm_l, sl_send, sl_recv, hop + 1, left)
            rd_l.start()
        out_ref[:, :HALF] += comm_r[hop + 1].astype(jnp.float32)
        out_ref[:, HALF:] += comm_l[hop + 1].astype(jnp.float32)


def kernel(x, Wq, Wo, Wk, Wv):
    x2 = x.reshape(SQ, D)
    out = pl.pallas_call(
        _body,
        out_shape=jax.ShapeDtypeStruct((SQ, D), jnp.float32),
        in_specs=[pl.BlockSpec(memory_space=pltpu.VMEM)] * 5,
        out_specs=pl.BlockSpec(memory_space=pltpu.VMEM),
        scratch_shapes=[
            pltpu.VMEM((SQ, D), jnp.bfloat16),
            pltpu.VMEM((N_DEV, SQ, HALF), jnp.bfloat16),
            pltpu.VMEM((N_DEV, SQ, HALF), jnp.bfloat16),
            pltpu.SemaphoreType.DMA((N_DEV - 1,)),
            pltpu.SemaphoreType.DMA((N_DEV - 1,)),
            pltpu.SemaphoreType.DMA((N_DEV - 1,)),
            pltpu.SemaphoreType.DMA((N_DEV - 1,)),
        ],
        compiler_params=pltpu.CompilerParams(collective_id=0),
    )(x2, Wq, Wo, Wk, Wv)
    return out.reshape(1, SQ, D)
